# baseline (device time: 38985 ns/iter reference)
import jax
import jax.numpy as jnp
from jax import lax
from jax.experimental import pallas as pl
from jax.experimental.pallas import tpu as pltpu

N_DEV = 8
B, SQ, D = 2, 128, 512
HQ_LOCAL, DH = 8, 64
M = B * SQ
CHUNK = M // N_DEV


def kernel(x, Wq, Wo, K_ext, V_ext):
    def body(x_ref, wq_ref, wo_ref, k_ref, v_ref, out_ref,
             x_v, wq_v, wo_v, k_v, v_v,
             rs_ref, ag_ref, send_ref, copy_sems,
             rs_send_sems, rs_recv_sems, ag_send_sems, ag_recv_sems):
        my_pos = lax.axis_index("i")
        h0 = my_pos * HQ_LOCAL

        cp_x = pltpu.make_async_copy(x_ref, x_v, copy_sems.at[0])
        cp_wq = pltpu.make_async_copy(wq_ref, wq_v, copy_sems.at[1])
        cp_wo = pltpu.make_async_copy(wo_ref, wo_v, copy_sems.at[2])
        cp_k = pltpu.make_async_copy(
            k_ref.at[:, :, pl.ds(h0, HQ_LOCAL), :], k_v, copy_sems.at[3])
        cp_v = pltpu.make_async_copy(
            v_ref.at[:, :, pl.ds(h0, HQ_LOCAL), :], v_v, copy_sems.at[4])
        for cp in (cp_x, cp_wq, cp_wo, cp_k, cp_v):
            cp.start()

        barrier_sem = pltpu.get_barrier_semaphore()
        for k in range(1, N_DEV):
            peer = lax.rem(my_pos + k, N_DEV)
            pl.semaphore_signal(
                barrier_sem, inc=1,
                device_id=(peer,), device_id_type=pl.DeviceIdType.MESH,
            )
        pl.semaphore_wait(barrier_sem, N_DEV - 1)

        cp_x.wait()
        cp_wq.wait()
        x2d = x_v[:].reshape(M, D)
        q = jnp.dot(x2d, wq_v[:], preferred_element_type=jnp.float32)

        cp_k.wait()
        cp_v.wait()
        rows = []
        for b in range(B):
            cols = []
            for h in range(HQ_LOCAL):
                qh = q[b * SQ:(b + 1) * SQ, h * DH:(h + 1) * DH]
                khh = k_v[b, :, h, :]
                vhh = v_v[b, :, h, :]
                s = lax.dot_general(
                    qh, khh, (((1,), (1,)), ((), ())),
                    preferred_element_type=jnp.float32,
                ) * 0.125
                mx = jnp.max(s, axis=-1, keepdims=True)
                p = jnp.exp(s - mx)
                l = jnp.sum(p, axis=-1, keepdims=True)
                cols.append(jnp.dot(p / l, vhh,
                                    preferred_element_type=jnp.float32))
            rows.append(jnp.concatenate(cols, axis=1))
        attn2d = jnp.concatenate(rows, axis=0)

        cp_wo.wait()
        partial = jnp.dot(attn2d, wo_v[:], preferred_element_type=jnp.float32)

        send_ref[:] = partial
        rs_ref[my_pos] = send_ref[pl.ds(my_pos * CHUNK, CHUNK), :]

        rs_sends = []
        for k in range(1, N_DEV):
            peer = lax.rem(my_pos + k, N_DEV)
            rdma = pltpu.make_async_remote_copy(
                src_ref=send_ref.at[pl.ds(peer * CHUNK, CHUNK), :],
                dst_ref=rs_ref.at[my_pos],
                send_sem=rs_send_sems.at[k],
                recv_sem=rs_recv_sems.at[my_pos],
                device_id=(peer,),
                device_id_type=pl.DeviceIdType.MESH,
            )
            rdma.start()
            rs_sends.append(rdma)

        for k in range(1, N_DEV):
            src_peer = lax.rem(my_pos + k, N_DEV)
            recv = pltpu.make_async_remote_copy(
                src_ref=send_ref.at[pl.ds(0, CHUNK), :],
                dst_ref=rs_ref.at[src_peer],
                send_sem=rs_send_sems.at[0],
                recv_sem=rs_recv_sems.at[src_peer],
                device_id=(src_peer,),
                device_id_type=pl.DeviceIdType.MESH,
            )
            recv.wait_recv()

        ag_ref[my_pos] = jnp.sum(rs_ref[:], axis=0)

        ag_sends = []
        for k in range(1, N_DEV):
            peer = lax.rem(my_pos + k, N_DEV)
            rdma = pltpu.make_async_remote_copy(
                src_ref=ag_ref.at[my_pos],
                dst_ref=ag_ref.at[my_pos],
                send_sem=ag_send_sems.at[k],
                recv_sem=ag_recv_sems.at[my_pos],
                device_id=(peer,),
                device_id_type=pl.DeviceIdType.MESH,
            )
            rdma.start()
            ag_sends.append(rdma)

        for k in range(1, N_DEV):
            src_peer = lax.rem(my_pos + k, N_DEV)
            recv = pltpu.make_async_remote_copy(
                src_ref=ag_ref.at[src_peer],
                dst_ref=ag_ref.at[src_peer],
                send_sem=ag_send_sems.at[0],
                recv_sem=ag_recv_sems.at[src_peer],
                device_id=(src_peer,),
                device_id_type=pl.DeviceIdType.MESH,
            )
            recv.wait_recv()

        out_ref[:] = ag_ref[:].reshape(B, SQ, D)

        for rdma in rs_sends + ag_sends:
            rdma.wait_send()

    return pl.pallas_call(
        body,
        out_shape=jax.ShapeDtypeStruct((B, SQ, D), jnp.float32),
        in_specs=[pl.BlockSpec(memory_space=pl.ANY)] * 5,
        out_specs=pl.BlockSpec(memory_space=pltpu.VMEM),
        scratch_shapes=[
            pltpu.VMEM((B, SQ, D), jnp.float32),
            pltpu.VMEM((D, D), jnp.float32),
            pltpu.VMEM((D, D), jnp.float32),
            pltpu.VMEM((B, SQ, HQ_LOCAL, DH), jnp.float32),
            pltpu.VMEM((B, SQ, HQ_LOCAL, DH), jnp.float32),
            pltpu.VMEM((N_DEV, CHUNK, D), jnp.float32),
            pltpu.VMEM((N_DEV, CHUNK, D), jnp.float32),
            pltpu.VMEM((M, D), jnp.float32),
            pltpu.SemaphoreType.DMA((5,)),
            pltpu.SemaphoreType.DMA((N_DEV,)),
            pltpu.SemaphoreType.DMA((N_DEV,)),
            pltpu.SemaphoreType.DMA((N_DEV,)),
            pltpu.SemaphoreType.DMA((N_DEV,)),
        ],
        compiler_params=pltpu.CompilerParams(collective_id=0),
    )(x, Wq, Wo, K_ext, V_ext)


# device time: 24360 ns/iter; 1.6004x vs baseline; 1.6004x over previous
import jax
import jax.numpy as jnp
from jax import lax
from jax.experimental import pallas as pl
from jax.experimental.pallas import tpu as pltpu

N_DEV = 8
B, SQ, D = 2, 128, 512
HQ_LOCAL, DH = 8, 64
M = B * SQ
CHUNK = M // N_DEV


def kernel(x, Wq, Wo, K_ext, V_ext):
    h0 = lax.axis_index("i") * HQ_LOCAL
    Kh = lax.dynamic_slice_in_dim(K_ext, h0, HQ_LOCAL, axis=2)
    Vh = lax.dynamic_slice_in_dim(V_ext, h0, HQ_LOCAL, axis=2)
    Kh = Kh.reshape(B, SQ, HQ_LOCAL * DH)
    Vh = Vh.reshape(B, SQ, HQ_LOCAL * DH)

    def body(x_ref, wq_ref, wo_ref, k_ref, v_ref, out_ref,
             x_v, wq_v, wo_v, k_v, v_v,
             rs_ref, ag_ref, send_ref, copy_sems,
             rs_send_sems, rs_recv_sems, ag_send_sems, ag_recv_sems):
        my_pos = lax.axis_index("i")

        cp_x = pltpu.make_async_copy(x_ref, x_v, copy_sems.at[0])
        cp_wq = pltpu.make_async_copy(wq_ref, wq_v, copy_sems.at[1])
        cp_wo = pltpu.make_async_copy(wo_ref, wo_v, copy_sems.at[2])
        cp_k = pltpu.make_async_copy(k_ref, k_v, copy_sems.at[3])
        cp_v = pltpu.make_async_copy(v_ref, v_v, copy_sems.at[4])
        for cp in (cp_x, cp_wq, cp_wo, cp_k, cp_v):
            cp.start()

        barrier_sem = pltpu.get_barrier_semaphore()
        for k in range(1, N_DEV):
            peer = lax.rem(my_pos + k, N_DEV)
            pl.semaphore_signal(
                barrier_sem, inc=1,
                device_id=(peer,), device_id_type=pl.DeviceIdType.MESH,
            )
        pl.semaphore_wait(barrier_sem, N_DEV - 1)

        cp_x.wait()
        cp_wq.wait()
        x2d = x_v[:].reshape(M, D)
        q = jnp.dot(x2d, wq_v[:], preferred_element_type=jnp.float32)

        cp_k.wait()
        cp_v.wait()
        rows = []
        for b in range(B):
            cols = []
            for h in range(HQ_LOCAL):
                qh = q[b * SQ:(b + 1) * SQ, h * DH:(h + 1) * DH]
                khh = k_v[b, :, h * DH:(h + 1) * DH]
                vhh = v_v[b, :, h * DH:(h + 1) * DH]
                s = lax.dot_general(
                    qh, khh, (((1,), (1,)), ((), ())),
                    preferred_element_type=jnp.float32,
                ) * 0.125
                mx = jnp.max(s, axis=-1, keepdims=True)
                p = jnp.exp(s - mx)
                l = jnp.sum(p, axis=-1, keepdims=True)
                cols.append(jnp.dot(p / l, vhh,
                                    preferred_element_type=jnp.float32))
            rows.append(jnp.concatenate(cols, axis=1))
        attn2d = jnp.concatenate(rows, axis=0)

        cp_wo.wait()
        partial = jnp.dot(attn2d, wo_v[:], preferred_element_type=jnp.float32)

        send_ref[:] = partial
        rs_ref[my_pos] = send_ref[pl.ds(my_pos * CHUNK, CHUNK), :]

        rs_sends = []
        for k in range(1, N_DEV):
            peer = lax.rem(my_pos + k, N_DEV)
            rdma = pltpu.make_async_remote_copy(
                src_ref=send_ref.at[pl.ds(peer * CHUNK, CHUNK), :],
                dst_ref=rs_ref.at[my_pos],
                send_sem=rs_send_sems.at[k],
                recv_sem=rs_recv_sems.at[my_pos],
                device_id=(peer,),
                device_id_type=pl.DeviceIdType.MESH,
            )
            rdma.start()
            rs_sends.append(rdma)

        for k in range(1, N_DEV):
            src_peer = lax.rem(my_pos + k, N_DEV)
            recv = pltpu.make_async_remote_copy(
                src_ref=send_ref.at[pl.ds(0, CHUNK), :],
                dst_ref=rs_ref.at[src_peer],
                send_sem=rs_send_sems.at[0],
                recv_sem=rs_recv_sems.at[src_peer],
                device_id=(src_peer,),
                device_id_type=pl.DeviceIdType.MESH,
            )
            recv.wait_recv()

        ag_ref[my_pos] = jnp.sum(rs_ref[:], axis=0)

        ag_sends = []
        for k in range(1, N_DEV):
            peer = lax.rem(my_pos + k, N_DEV)
            rdma = pltpu.make_async_remote_copy(
                src_ref=ag_ref.at[my_pos],
                dst_ref=ag_ref.at[my_pos],
                send_sem=ag_send_sems.at[k],
                recv_sem=ag_recv_sems.at[my_pos],
                device_id=(peer,),
                device_id_type=pl.DeviceIdType.MESH,
            )
            rdma.start()
            ag_sends.append(rdma)

        for k in range(1, N_DEV):
            src_peer = lax.rem(my_pos + k, N_DEV)
            recv = pltpu.make_async_remote_copy(
                src_ref=ag_ref.at[src_peer],
                dst_ref=ag_ref.at[src_peer],
                send_sem=ag_send_sems.at[0],
                recv_sem=ag_recv_sems.at[src_peer],
                device_id=(src_peer,),
                device_id_type=pl.DeviceIdType.MESH,
            )
            recv.wait_recv()

        out_ref[:] = ag_ref[:].reshape(B, SQ, D)

        for rdma in rs_sends + ag_sends:
            rdma.wait_send()

    return pl.pallas_call(
        body,
        out_shape=jax.ShapeDtypeStruct((B, SQ, D), jnp.float32),
        in_specs=[pl.BlockSpec(memory_space=pl.ANY)] * 5,
        out_specs=pl.BlockSpec(memory_space=pltpu.VMEM),
        scratch_shapes=[
            pltpu.VMEM((B, SQ, D), jnp.float32),
            pltpu.VMEM((D, D), jnp.float32),
            pltpu.VMEM((D, D), jnp.float32),
            pltpu.VMEM((B, SQ, HQ_LOCAL * DH), jnp.float32),
            pltpu.VMEM((B, SQ, HQ_LOCAL * DH), jnp.float32),
            pltpu.VMEM((N_DEV, CHUNK, D), jnp.float32),
            pltpu.VMEM((N_DEV, CHUNK, D), jnp.float32),
            pltpu.VMEM((M, D), jnp.float32),
            pltpu.SemaphoreType.DMA((5,)),
            pltpu.SemaphoreType.DMA((N_DEV,)),
            pltpu.SemaphoreType.DMA((N_DEV,)),
            pltpu.SemaphoreType.DMA((N_DEV,)),
            pltpu.SemaphoreType.DMA((N_DEV,)),
        ],
        compiler_params=pltpu.CompilerParams(collective_id=0),
    )(x, Wq, Wo, Kh, Vh)
